# baseline (speedup 1.0000x reference)
import jax
import jax.numpy as jnp
from jax import lax
from jax.experimental import pallas as pl
from jax.experimental.pallas import tpu as pltpu
from jax.experimental.pallas import tpu_sc as plsc

N = 10000
E = 320000
ND = 128
XD = 16

NC = 2
NS = 16
NW = NC * NS

CH = 80
CHS = 40
NBUF = 5
G_PER_W = (2 * E) // NW
G_STEPS = G_PER_W // CH
G_GROUPS = G_STEPS // NBUF
S_PER_W = E // NW
S_STEPS = S_PER_W // CHS
S_GROUPS = S_STEPS // NBUF

ROWS_PER_TILE = N // NS


def _silu(v):
    return v * jax.nn.sigmoid(v)


def _mesh():
    return plsc.VectorSubcoreMesh(core_axis_name="c", subcore_axis_name="s",
                                  num_cores=NC, num_subcores=NS)


def _stage_rows(src_hbm, dst_sh, row0, bounce, cap):
    n_full = ROWS_PER_TILE // cap
    rem = ROWS_PER_TILE - n_full * cap
    for j in range(n_full + (1 if rem else 0)):
        sz = cap if j < n_full else rem
        r = row0 + j * cap
        pltpu.sync_copy(src_hbm.at[pl.ds(r, sz)], bounce.at[pl.ds(0, sz)])
        pltpu.sync_copy(bounce.at[pl.ds(0, sz)], dst_sh.at[pl.ds(r, sz)])


def _gather_body(h_hbm, tx_hbm, idx_hbm, gh_hbm, gx_hbm,
                 idx_all, hbs, xbs, gsh, gsx, osh, osx):
    cid = lax.axis_index("c")
    sid = lax.axis_index("s")
    wid = sid * NC + cid
    w_base = wid * G_PER_W

    w_step0 = wid * G_STEPS
    pltpu.sync_copy(idx_hbm.at[pl.ds(w_step0, G_STEPS)], idx_all)
    plsc.subcore_barrier()

    def group(g, carry):
        k0 = g * NBUF
        for b in range(NBUF):
            k = k0 + b
            pltpu.async_copy(h_hbm.at[idx_all.at[k]], hbs[b], gsh[b])
            pltpu.async_copy(tx_hbm.at[idx_all.at[k]], xbs[b], gsx[b])
        for b in range(NBUF):
            k = k0 + b
            base = w_base + k * CH
            pltpu.make_async_copy(h_hbm.at[idx_all.at[k]], hbs[b],
                                  gsh[b]).wait()
            pltpu.make_async_copy(tx_hbm.at[idx_all.at[k]], xbs[b],
                                  gsx[b]).wait()
            pltpu.async_copy(hbs[b], gh_hbm.at[pl.ds(base, CH)], osh[b])
            pltpu.async_copy(xbs[b], gx_hbm.at[pl.ds(base, CH)], osx[b])
        for b in range(NBUF):
            base = w_base + (k0 + b) * CH
            pltpu.make_async_copy(hbs[b], gh_hbm.at[pl.ds(base, CH)],
                                  osh[b]).wait()
            pltpu.make_async_copy(xbs[b], gx_hbm.at[pl.ds(base, CH)],
                                  osx[b]).wait()
        return carry

    lax.fori_loop(0, G_GROUPS, group, 0)


def _sc_gather(h, tx, flat_idx):
    scratch = [
        pltpu.VMEM((G_STEPS, CH), jnp.int32),
        [pltpu.VMEM((CH, ND), jnp.bfloat16) for _ in range(NBUF)],
        [pltpu.VMEM((CH, XD), jnp.float32) for _ in range(NBUF)],
        [pltpu.SemaphoreType.DMA for _ in range(NBUF)],
        [pltpu.SemaphoreType.DMA for _ in range(NBUF)],
        [pltpu.SemaphoreType.DMA for _ in range(NBUF)],
        [pltpu.SemaphoreType.DMA for _ in range(NBUF)],
    ]
    return pl.kernel(
        _gather_body,
        out_type=(
            jax.ShapeDtypeStruct((2 * E, ND), jnp.bfloat16),
            jax.ShapeDtypeStruct((2 * E, XD), jnp.float32),
        ),
        mesh=_mesh(),
        scratch_types=scratch,
        compiler_params=pltpu.CompilerParams(use_tc_tiling_on_sc=False),
    )(h, tx, flat_idx)


def _scatter_body(m_hbm, cw_hbm, dst_hbm, zh_hbm, zx_hbm, ph_hbm, px_hbm,
                  acc_h, acc_x, idx_all, mbs, cbs, gsm, gsc):
    cid = lax.axis_index("c")
    sid = lax.axis_index("s")
    wid = sid * NC + cid
    w_base = wid * S_PER_W
    row0 = sid * ROWS_PER_TILE

    _stage_rows(zh_hbm, acc_h, row0, mbs[0], CHS)
    _stage_rows(zx_hbm, acc_x, row0, cbs[0], CHS)
    w_step0 = wid * S_STEPS
    pltpu.sync_copy(dst_hbm.at[pl.ds(w_step0, S_STEPS)], idx_all)
    plsc.subcore_barrier()

    def group(g, carry):
        k0 = g * NBUF
        for b in range(NBUF):
            base = w_base + (k0 + b) * CHS
            pltpu.async_copy(m_hbm.at[pl.ds(base, CHS)], mbs[b], gsm[b])
            pltpu.async_copy(cw_hbm.at[pl.ds(base, CHS)], cbs[b], gsc[b])
        for b in range(NBUF):
            k = k0 + b
            base = w_base + k * CHS
            pltpu.make_async_copy(m_hbm.at[pl.ds(base, CHS)], mbs[b],
                                  gsm[b]).wait()
            pltpu.make_async_copy(cw_hbm.at[pl.ds(base, CHS)], cbs[b],
                                  gsc[b]).wait()
            pltpu.sync_copy(mbs[b], acc_h.at[idx_all.at[k]], add=True)
            pltpu.sync_copy(cbs[b], acc_x.at[idx_all.at[k]], add=True)
        return carry

    lax.fori_loop(0, S_GROUPS, group, 0)
    plsc.subcore_barrier()

    n_full = ROWS_PER_TILE // CHS
    rem = ROWS_PER_TILE - n_full * CHS
    for j in range(n_full + (1 if rem else 0)):
        sz = CHS if j < n_full else rem
        r = row0 + j * CHS
        pltpu.sync_copy(acc_h.at[pl.ds(r, sz)], mbs[0].at[pl.ds(0, sz)])
        pltpu.sync_copy(mbs[0].at[pl.ds(0, sz)], ph_hbm.at[cid, pl.ds(r, sz)])
        pltpu.sync_copy(acc_x.at[pl.ds(r, sz)], cbs[0].at[pl.ds(0, sz)])
        pltpu.sync_copy(cbs[0].at[pl.ds(0, sz)], px_hbm.at[cid, pl.ds(r, sz)])


def _sc_scatter(m, cwdir, dst, zh, zx):
    scratch = [
        pltpu.VMEM_SHARED((N, ND), jnp.float32),
        pltpu.VMEM_SHARED((N, XD), jnp.float32),
        pltpu.VMEM((S_STEPS, CHS), jnp.int32),
        [pltpu.VMEM((CHS, ND), jnp.float32) for _ in range(NBUF)],
        [pltpu.VMEM((CHS, XD), jnp.float32) for _ in range(NBUF)],
        [pltpu.SemaphoreType.DMA for _ in range(NBUF)],
        [pltpu.SemaphoreType.DMA for _ in range(NBUF)],
    ]
    return pl.kernel(
        _scatter_body,
        out_type=(
            jax.ShapeDtypeStruct((NC, N, ND), jnp.float32),
            jax.ShapeDtypeStruct((NC, N, XD), jnp.float32),
        ),
        mesh=_mesh(),
        scratch_types=scratch,
        compiler_params=pltpu.CompilerParams(use_tc_tiling_on_sc=False),
    )(m, cwdir, dst, zh, zx)


BE = 512


def _edge_body(dist_ref, hs_ref, hd_ref, xs_ref, xd_ref,
               whs_ref, whd_ref, we_ref, b1_ref, w2n_ref, b2n_ref, w2c_ref,
               ew1t_ref, eb1t_ref, ew2_ref, eb2t_ref,
               m_ref, cw_ref):
    pr = lax.Precision.DEFAULT
    distrow = dist_ref[0]
    pe = _silu(ew1t_ref[...] * distrow + eb1t_ref[...])
    eat = lax.dot_general(ew2_ref[...], pe, (((0,), (0,)), ((), ())),
                          precision=pr, preferred_element_type=jnp.float32)
    eat = eat + eb2t_ref[...]
    pre = lax.dot_general(hs_ref[...], whs_ref[...], (((1,), (0,)), ((), ())),
                          precision=pr, preferred_element_type=jnp.float32)
    pre = pre + lax.dot_general(hd_ref[...], whd_ref[...],
                                (((1,), (0,)), ((), ())),
                                precision=pr, preferred_element_type=jnp.float32)
    pre = pre + lax.dot_general(eat, we_ref[...], (((0,), (0,)), ((), ())),
                                precision=pr, preferred_element_type=jnp.float32)
    act = _silu(pre + b1_ref[...])
    act16 = act.astype(jnp.bfloat16)
    m = lax.dot_general(act16[:, :ND], w2n_ref[...], (((1,), (0,)), ((), ())),
                        precision=pr, preferred_element_type=jnp.float32)
    m_ref[...] = m + b2n_ref[...]
    cw = lax.dot_general(act16[:, ND:], w2c_ref[...], (((1,), (0,)), ((), ())),
                         precision=pr, preferred_element_type=jnp.float32)
    dxyz = xs_ref[...] - xd_ref[...]
    l2 = jnp.sum(dxyz * dxyz, axis=1, keepdims=True)
    ln = jnp.maximum(jnp.sqrt(l2), 1e-8)
    cw_ref[...] = dxyz * (cw / ln)


def _tc_edges(distrows, gh, gx, whs, whd, we, b1, w2n, b2n, w2c,
              ew1t, eb1t, ew2, eb2t):
    grid = (E // BE,)
    full = lambda shape: pl.BlockSpec(shape, lambda i: (0, 0))
    return pl.pallas_call(
        _edge_body,
        grid=grid,
        in_specs=[
            pl.BlockSpec((1, 1, BE), lambda i: (i, 0, 0)),
            pl.BlockSpec((BE, ND), lambda i: (i, 0)),
            pl.BlockSpec((BE, ND), lambda i: (E // BE + i, 0)),
            pl.BlockSpec((BE, XD), lambda i: (i, 0)),
            pl.BlockSpec((BE, XD), lambda i: (E // BE + i, 0)),
            full((ND, 2 * ND)),
            full((ND, 2 * ND)),
            full((16, 2 * ND)),
            full((1, 2 * ND)),
            full((ND, ND)),
            full((1, ND)),
            full((ND, 1)),
            full((16, 1)),
            full((16, 1)),
            full((16, 16)),
            full((16, 1)),
        ],
        out_specs=[
            pl.BlockSpec((BE, ND), lambda i: (i, 0)),
            pl.BlockSpec((BE, XD), lambda i: (i, 0)),
        ],
        out_shape=[
            jax.ShapeDtypeStruct((E, ND), jnp.float32),
            jax.ShapeDtypeStruct((E, XD), jnp.float32),
        ],
        compiler_params=pltpu.CompilerParams(
            dimension_semantics=("arbitrary",)),
    )(distrows, gh, gh, gx, gx, whs, whd, we, b1, w2n, b2n, w2c,
      ew1t, eb1t, ew2, eb2t)


BN = 2000


def _combine_body(h_ref, tx_ref, pha_ref, phb_ref, pxa_ref, pxb_ref,
                  oh_ref, ox_ref):
    oh_ref[...] = h_ref[...] + pha_ref[0] + phb_ref[0]
    ox_ref[...] = tx_ref[...] + pxa_ref[0] + pxb_ref[0]


def _tc_combine(h, tx, ph, px):
    grid = (N // BN,)
    return pl.pallas_call(
        _combine_body,
        grid=grid,
        in_specs=[
            pl.BlockSpec((BN, ND), lambda i: (i, 0)),
            pl.BlockSpec((BN, XD), lambda i: (i, 0)),
            pl.BlockSpec((1, BN, ND), lambda i: (0, i, 0)),
            pl.BlockSpec((1, BN, ND), lambda i: (1, i, 0)),
            pl.BlockSpec((1, BN, XD), lambda i: (0, i, 0)),
            pl.BlockSpec((1, BN, XD), lambda i: (1, i, 0)),
        ],
        out_specs=[
            pl.BlockSpec((BN, ND), lambda i: (i, 0)),
            pl.BlockSpec((BN, XD), lambda i: (i, 0)),
        ],
        out_shape=[
            jax.ShapeDtypeStruct((N, ND), jnp.float32),
            jax.ShapeDtypeStruct((N, XD), jnp.float32),
        ],
        compiler_params=pltpu.CompilerParams(
            dimension_semantics=("arbitrary",)),
    )(h, tx, ph, ph, px, px)


@jax.jit
def kernel(h, x, edge_idx, edge_dist,
           node_W1, node_b1, node_W2, node_b2,
           coord_W1, coord_b1, coord_W2,
           edge_W1, edge_b1, edge_W2, edge_b2):
    f32 = jnp.float32
    tx = jnp.pad(x, ((0, 0), (0, XD - 3)))
    flat_idx = edge_idx.reshape(2 * E // CH, CH)
    dst = edge_idx[1].reshape(E // CHS, CHS)
    distrows = edge_dist.reshape(E // BE, 1, BE)

    whs = jnp.concatenate([node_W1[:ND], coord_W1[:ND]],
                          axis=1).astype(jnp.bfloat16)
    whd = jnp.concatenate([node_W1[ND:2 * ND], coord_W1[ND:2 * ND]],
                          axis=1).astype(jnp.bfloat16)
    we = jnp.concatenate([node_W1[2 * ND:], coord_W1[2 * ND:]], axis=1)
    b1 = jnp.concatenate([node_b1, coord_b1]).reshape(1, 2 * ND)
    b2n = node_b2.reshape(1, ND)
    ew1t = edge_W1.reshape(16, 1)
    eb1t = edge_b1.reshape(16, 1)
    eb2t = edge_b2.reshape(16, 1)
    zh = jnp.zeros((N, ND), f32)
    zx = jnp.zeros((N, XD), f32)

    h16 = h.astype(jnp.bfloat16)
    gh, gx = _sc_gather(h16, tx, flat_idx)
    m, cwdir = _tc_edges(distrows, gh, gx, whs, whd, we, b1,
                         node_W2.astype(jnp.bfloat16), b2n,
                         coord_W2.astype(jnp.bfloat16), ew1t, eb1t,
                         edge_W2, eb2t)
    ph, px = _sc_scatter(m, cwdir, dst, zh, zx)
    oh, ox = _tc_combine(h, tx, ph, px)
    return oh, ox[:, :3]


# speedup vs baseline: 1.7285x; 1.7285x over previous
import jax
import jax.numpy as jnp
from jax import lax
from jax.experimental import pallas as pl
from jax.experimental.pallas import tpu as pltpu
from jax.experimental.pallas import tpu_sc as plsc

N = 10000
E = 320000
ND = 128
XD = 16

NC = 2
NS = 16
NW = NC * NS

CH = 80
CHS = 40
NBUF = 5
G_PER_W = (2 * E) // NW
G_STEPS = G_PER_W // CH
G_GROUPS = G_STEPS // NBUF
S_PER_W = E // NW
S_STEPS = S_PER_W // CHS
S_GROUPS = S_STEPS // NBUF

ROWS_PER_TILE = N // NS


def _silu(v):
    return v * jax.nn.sigmoid(v)


def _mesh():
    return plsc.VectorSubcoreMesh(core_axis_name="c", subcore_axis_name="s",
                                  num_cores=NC, num_subcores=NS)


def _stage_rows(src_hbm, dst_sh, row0, bounce, cap):
    n_full = ROWS_PER_TILE // cap
    rem = ROWS_PER_TILE - n_full * cap
    for j in range(n_full + (1 if rem else 0)):
        sz = cap if j < n_full else rem
        r = row0 + j * cap
        pltpu.sync_copy(src_hbm.at[pl.ds(r, sz)], bounce.at[pl.ds(0, sz)])
        pltpu.sync_copy(bounce.at[pl.ds(0, sz)], dst_sh.at[pl.ds(r, sz)])


def _gather_body(h_hbm, tx_hbm, idx_hbm, gh_hbm, gx_hbm,
                 idx_all, hbs, xbs, gsh, gsx, osh, osx):
    cid = lax.axis_index("c")
    sid = lax.axis_index("s")
    wid = sid * NC + cid
    w_base = wid * G_PER_W

    w_step0 = wid * G_STEPS
    pltpu.sync_copy(idx_hbm.at[pl.ds(w_step0, G_STEPS)], idx_all)
    plsc.subcore_barrier()

    def group(g, carry):
        k0 = g * NBUF
        for b in range(NBUF):
            k = k0 + b
            pltpu.async_copy(h_hbm.at[idx_all.at[k]], hbs[b], gsh[b])
            pltpu.async_copy(tx_hbm.at[idx_all.at[k]], xbs[b], gsx[b])
        for b in range(NBUF):
            k = k0 + b
            base = w_base + k * CH
            pltpu.make_async_copy(h_hbm.at[idx_all.at[k]], hbs[b],
                                  gsh[b]).wait()
            pltpu.make_async_copy(tx_hbm.at[idx_all.at[k]], xbs[b],
                                  gsx[b]).wait()
            pltpu.async_copy(hbs[b], gh_hbm.at[pl.ds(base, CH)], osh[b])
            pltpu.async_copy(xbs[b], gx_hbm.at[pl.ds(base, CH)], osx[b])
        for b in range(NBUF):
            base = w_base + (k0 + b) * CH
            pltpu.make_async_copy(hbs[b], gh_hbm.at[pl.ds(base, CH)],
                                  osh[b]).wait()
            pltpu.make_async_copy(xbs[b], gx_hbm.at[pl.ds(base, CH)],
                                  osx[b]).wait()
        return carry

    lax.fori_loop(0, G_GROUPS, group, 0)


def _sc_gather(h, tx, flat_idx):
    scratch = [
        pltpu.VMEM((G_STEPS, CH), jnp.int32),
        [pltpu.VMEM((CH, ND), jnp.float32) for _ in range(NBUF)],
        [pltpu.VMEM((CH, XD), jnp.float32) for _ in range(NBUF)],
        [pltpu.SemaphoreType.DMA for _ in range(NBUF)],
        [pltpu.SemaphoreType.DMA for _ in range(NBUF)],
        [pltpu.SemaphoreType.DMA for _ in range(NBUF)],
        [pltpu.SemaphoreType.DMA for _ in range(NBUF)],
    ]
    return pl.kernel(
        _gather_body,
        out_type=(
            jax.ShapeDtypeStruct((2 * E, ND), jnp.float32),
            jax.ShapeDtypeStruct((2 * E, XD), jnp.float32),
        ),
        mesh=_mesh(),
        scratch_types=scratch,
        compiler_params=pltpu.CompilerParams(use_tc_tiling_on_sc=False),
    )(h, tx, flat_idx)


def _scatter_body(m_hbm, cw_hbm, dst_hbm, zh_hbm, zx_hbm, ph_hbm, px_hbm,
                  acc_h, acc_x, idx_all, mbs, cbs, gsm, gsc):
    cid = lax.axis_index("c")
    sid = lax.axis_index("s")
    wid = sid * NC + cid
    w_base = wid * S_PER_W
    row0 = sid * ROWS_PER_TILE

    _stage_rows(zh_hbm, acc_h, row0, mbs[0], CHS)
    _stage_rows(zx_hbm, acc_x, row0, cbs[0], CHS)
    w_step0 = wid * S_STEPS
    pltpu.sync_copy(dst_hbm.at[pl.ds(w_step0, S_STEPS)], idx_all)
    plsc.subcore_barrier()

    def group(g, carry):
        k0 = g * NBUF
        for b in range(NBUF):
            base = w_base + (k0 + b) * CHS
            pltpu.async_copy(m_hbm.at[pl.ds(base, CHS)], mbs[b], gsm[b])
            pltpu.async_copy(cw_hbm.at[pl.ds(base, CHS)], cbs[b], gsc[b])
        for b in range(NBUF):
            k = k0 + b
            base = w_base + k * CHS
            pltpu.make_async_copy(m_hbm.at[pl.ds(base, CHS)], mbs[b],
                                  gsm[b]).wait()
            pltpu.make_async_copy(cw_hbm.at[pl.ds(base, CHS)], cbs[b],
                                  gsc[b]).wait()
            pltpu.sync_copy(mbs[b], acc_h.at[idx_all.at[k]], add=True)
            pltpu.sync_copy(cbs[b], acc_x.at[idx_all.at[k]], add=True)
        return carry

    lax.fori_loop(0, S_GROUPS, group, 0)
    plsc.subcore_barrier()

    n_full = ROWS_PER_TILE // CHS
    rem = ROWS_PER_TILE - n_full * CHS
    for j in range(n_full + (1 if rem else 0)):
        sz = CHS if j < n_full else rem
        r = row0 + j * CHS
        pltpu.sync_copy(acc_h.at[pl.ds(r, sz)], mbs[0].at[pl.ds(0, sz)])
        pltpu.sync_copy(mbs[0].at[pl.ds(0, sz)], ph_hbm.at[cid, pl.ds(r, sz)])
        pltpu.sync_copy(acc_x.at[pl.ds(r, sz)], cbs[0].at[pl.ds(0, sz)])
        pltpu.sync_copy(cbs[0].at[pl.ds(0, sz)], px_hbm.at[cid, pl.ds(r, sz)])


def _sc_scatter(m, cwdir, dst, zh, zx):
    scratch = [
        pltpu.VMEM_SHARED((N, ND), jnp.float32),
        pltpu.VMEM_SHARED((N, XD), jnp.float32),
        pltpu.VMEM((S_STEPS, CHS), jnp.int32),
        [pltpu.VMEM((CHS, ND), jnp.float32) for _ in range(NBUF)],
        [pltpu.VMEM((CHS, XD), jnp.float32) for _ in range(NBUF)],
        [pltpu.SemaphoreType.DMA for _ in range(NBUF)],
        [pltpu.SemaphoreType.DMA for _ in range(NBUF)],
    ]
    return pl.kernel(
        _scatter_body,
        out_type=(
            jax.ShapeDtypeStruct((NC, N, ND), jnp.float32),
            jax.ShapeDtypeStruct((NC, N, XD), jnp.float32),
        ),
        mesh=_mesh(),
        scratch_types=scratch,
        compiler_params=pltpu.CompilerParams(use_tc_tiling_on_sc=False),
    )(m, cwdir, dst, zh, zx)


BE = 512


def _edge_body(dist_ref, hs_ref, hd_ref, xs_ref, xd_ref,
               whs_ref, whd_ref, we_ref, b1_ref, w2n_ref, b2n_ref, w2cb_ref,
               sel_ref, selt_ref,
               ew1t_ref, eb1t_ref, ew2_ref, eb2t_ref,
               m_ref, cw_ref):
    pr = lax.Precision.DEFAULT
    distrow = dist_ref[0]
    pe = _silu(ew1t_ref[...] * distrow + eb1t_ref[...])
    eat = lax.dot_general(ew2_ref[...], pe, (((0,), (0,)), ((), ())),
                          precision=pr, preferred_element_type=jnp.float32)
    eat = eat + eb2t_ref[...]
    pre = lax.dot_general(hs_ref[...], whs_ref[...], (((1,), (0,)), ((), ())),
                          precision=pr, preferred_element_type=jnp.float32)
    pre = pre + lax.dot_general(hd_ref[...], whd_ref[...],
                                (((1,), (0,)), ((), ())),
                                precision=pr, preferred_element_type=jnp.float32)
    pre = pre + lax.dot_general(eat, we_ref[...], (((0,), (0,)), ((), ())),
                                precision=pr, preferred_element_type=jnp.float32)
    act = _silu(pre + b1_ref[...])
    m = lax.dot_general(act[:, :ND], w2n_ref[...], (((1,), (0,)), ((), ())),
                        precision=pr, preferred_element_type=jnp.float32)
    m_ref[...] = m + b2n_ref[...]
    actc = act[:, ND:].reshape(BE // 8, 8 * ND)
    cwg = lax.dot_general(actc, w2cb_ref[...], (((1,), (0,)), ((), ())),
                          precision=pr, preferred_element_type=jnp.float32)
    dP = xs_ref[...] - xd_ref[...]
    l2g = lax.dot_general(dP * dP, sel_ref[...], (((1,), (0,)), ((), ())),
                          precision=pr, preferred_element_type=jnp.float32)
    lng = jnp.maximum(jnp.sqrt(l2g), 1e-8)
    scaleP = lax.dot_general(cwg / lng, selt_ref[...],
                             (((1,), (0,)), ((), ())),
                             precision=pr, preferred_element_type=jnp.float32)
    cw_ref[...] = dP * scaleP


def _tc_edges(distrows, gh, gxp, whs, whd, we, b1, w2n, b2n, w2cb, sel,
              selt, ew1t, eb1t, ew2, eb2t):
    grid = (E // BE,)
    full = lambda shape: pl.BlockSpec(shape, lambda i: (0, 0))
    return pl.pallas_call(
        _edge_body,
        grid=grid,
        in_specs=[
            pl.BlockSpec((1, 1, BE), lambda i: (i, 0, 0)),
            pl.BlockSpec((BE, ND), lambda i: (i, 0)),
            pl.BlockSpec((BE, ND), lambda i: (E // BE + i, 0)),
            pl.BlockSpec((BE // 8, ND), lambda i: (i, 0)),
            pl.BlockSpec((BE // 8, ND), lambda i: (E // BE + i, 0)),
            full((ND, 2 * ND)),
            full((ND, 2 * ND)),
            full((16, 2 * ND)),
            full((1, 2 * ND)),
            full((ND, ND)),
            full((1, ND)),
            full((8 * ND, 8)),
            full((ND, 8)),
            full((8, ND)),
            full((16, 1)),
            full((16, 1)),
            full((16, 16)),
            full((16, 1)),
        ],
        out_specs=[
            pl.BlockSpec((BE, ND), lambda i: (i, 0)),
            pl.BlockSpec((BE // 8, ND), lambda i: (i, 0)),
        ],
        out_shape=[
            jax.ShapeDtypeStruct((E, ND), jnp.float32),
            jax.ShapeDtypeStruct((E // 8, ND), jnp.float32),
        ],
        compiler_params=pltpu.CompilerParams(
            dimension_semantics=("arbitrary",)),
    )(distrows, gh, gh, gxp, gxp, whs, whd, we, b1, w2n, b2n, w2cb, sel, selt,
      ew1t, eb1t, ew2, eb2t)


BN = 2000


def _combine_body(h_ref, tx_ref, pha_ref, phb_ref, pxa_ref, pxb_ref,
                  oh_ref, ox_ref):
    oh_ref[...] = h_ref[...] + pha_ref[0] + phb_ref[0]
    ox_ref[...] = tx_ref[...] + pxa_ref[0] + pxb_ref[0]


def _tc_combine(h, tx, ph, px):
    grid = (N // BN,)
    return pl.pallas_call(
        _combine_body,
        grid=grid,
        in_specs=[
            pl.BlockSpec((BN, ND), lambda i: (i, 0)),
            pl.BlockSpec((BN, XD), lambda i: (i, 0)),
            pl.BlockSpec((1, BN, ND), lambda i: (0, i, 0)),
            pl.BlockSpec((1, BN, ND), lambda i: (1, i, 0)),
            pl.BlockSpec((1, BN, XD), lambda i: (0, i, 0)),
            pl.BlockSpec((1, BN, XD), lambda i: (1, i, 0)),
        ],
        out_specs=[
            pl.BlockSpec((BN, ND), lambda i: (i, 0)),
            pl.BlockSpec((BN, XD), lambda i: (i, 0)),
        ],
        out_shape=[
            jax.ShapeDtypeStruct((N, ND), jnp.float32),
            jax.ShapeDtypeStruct((N, XD), jnp.float32),
        ],
        compiler_params=pltpu.CompilerParams(
            dimension_semantics=("arbitrary",)),
    )(h, tx, ph, ph, px, px)


@jax.jit
def kernel(h, x, edge_idx, edge_dist,
           node_W1, node_b1, node_W2, node_b2,
           coord_W1, coord_b1, coord_W2,
           edge_W1, edge_b1, edge_W2, edge_b2):
    f32 = jnp.float32
    tx = jnp.pad(x, ((0, 0), (0, XD - 3)))
    flat_idx = edge_idx.reshape(2 * E // CH, CH)
    dst = edge_idx[1].reshape(E // CHS, CHS)
    distrows = edge_dist.reshape(E // BE, 1, BE)

    whs = jnp.concatenate([node_W1[:ND], coord_W1[:ND]], axis=1)
    whd = jnp.concatenate([node_W1[ND:2 * ND], coord_W1[ND:2 * ND]], axis=1)
    we = jnp.concatenate([node_W1[2 * ND:], coord_W1[2 * ND:]], axis=1)
    b1 = jnp.concatenate([node_b1, coord_b1]).reshape(1, 2 * ND)
    b2n = node_b2.reshape(1, ND)
    ew1t = edge_W1.reshape(16, 1)
    eb1t = edge_b1.reshape(16, 1)
    eb2t = edge_b2.reshape(16, 1)
    zh = jnp.zeros((N, ND), f32)
    zx = jnp.zeros((N, XD), f32)

    w2cb = jnp.kron(jnp.eye(8, dtype=f32), coord_W2)
    lane_grp = jnp.arange(ND, dtype=jnp.int32) // XD
    sel = (lane_grp[:, None] == jnp.arange(8)[None, :]).astype(f32)
    selt = sel.T

    gh, gx = _sc_gather(h, tx, flat_idx)
    gxp = gx.reshape(2 * E // 8, ND)
    m, cwp = _tc_edges(distrows, gh, gxp, whs, whd, we, b1,
                       node_W2, b2n, w2cb, sel, selt, ew1t, eb1t,
                       edge_W2, eb2t)
    cwdir = cwp.reshape(E, XD)
    ph, px = _sc_scatter(m, cwdir, dst, zh, zx)
    oh, ox = _tc_combine(h, tx, ph, px)
    return oh, ox[:, :3]


# speedup vs baseline: 2.3026x; 1.3321x over previous
import jax
import jax.numpy as jnp
from jax import lax
from jax.experimental import pallas as pl
from jax.experimental.pallas import tpu as pltpu
from jax.experimental.pallas import tpu_sc as plsc

N = 10000
E = 320000
ND = 128
XD = 16

NC = 2
NS = 16
NW = NC * NS

CH = 80
CHS = 40
NBUF = 5
G_PER_W = (2 * E) // NW
G_STEPS = G_PER_W // CH
G_GROUPS = G_STEPS // NBUF
S_PER_W = E // NW
S_STEPS = S_PER_W // CHS
S_GROUPS = S_STEPS // NBUF

ROWS_PER_TILE = N // NS


def _silu(v):
    return v * jax.nn.sigmoid(v)


def _mesh():
    return plsc.VectorSubcoreMesh(core_axis_name="c", subcore_axis_name="s",
                                  num_cores=NC, num_subcores=NS)


def _stage_rows(src_hbm, dst_sh, row0, bounce, cap):
    n_full = ROWS_PER_TILE // cap
    rem = ROWS_PER_TILE - n_full * cap
    for j in range(n_full + (1 if rem else 0)):
        sz = cap if j < n_full else rem
        r = row0 + j * cap
        pltpu.sync_copy(src_hbm.at[pl.ds(r, sz)], bounce.at[pl.ds(0, sz)])
        pltpu.sync_copy(bounce.at[pl.ds(0, sz)], dst_sh.at[pl.ds(r, sz)])


def _gather_body(h_hbm, tx_hbm, idx_hbm, gh_hbm, gx_hbm,
                 idx_all, hbs, xbs, gsh, gsx, osh, osx):
    cid = lax.axis_index("c")
    sid = lax.axis_index("s")
    wid = sid * NC + cid
    w_base = wid * G_PER_W

    w_step0 = wid * G_STEPS
    pltpu.sync_copy(idx_hbm.at[pl.ds(w_step0, G_STEPS)], idx_all)
    plsc.subcore_barrier()

    def group(g, carry):
        k0 = g * NBUF
        for b in range(NBUF):
            k = k0 + b
            pltpu.async_copy(h_hbm.at[idx_all.at[k]], hbs[b], gsh[b])
            pltpu.async_copy(tx_hbm.at[idx_all.at[k]], xbs[b], gsx[b])
        for b in range(NBUF):
            k = k0 + b
            base = w_base + k * CH
            pltpu.make_async_copy(h_hbm.at[idx_all.at[k]], hbs[b],
                                  gsh[b]).wait()
            pltpu.make_async_copy(tx_hbm.at[idx_all.at[k]], xbs[b],
                                  gsx[b]).wait()
            pltpu.async_copy(hbs[b], gh_hbm.at[pl.ds(base, CH)], osh[b])
            pltpu.async_copy(xbs[b], gx_hbm.at[pl.ds(base, CH)], osx[b])
        for b in range(NBUF):
            base = w_base + (k0 + b) * CH
            pltpu.make_async_copy(hbs[b], gh_hbm.at[pl.ds(base, CH)],
                                  osh[b]).wait()
            pltpu.make_async_copy(xbs[b], gx_hbm.at[pl.ds(base, CH)],
                                  osx[b]).wait()
        return carry

    lax.fori_loop(0, G_GROUPS, group, 0)


def _sc_gather(h, tx, flat_idx):
    scratch = [
        pltpu.VMEM((G_STEPS, CH), jnp.int32),
        [pltpu.VMEM((CH, ND), jnp.float32) for _ in range(NBUF)],
        [pltpu.VMEM((CH, XD), jnp.float32) for _ in range(NBUF)],
        [pltpu.SemaphoreType.DMA for _ in range(NBUF)],
        [pltpu.SemaphoreType.DMA for _ in range(NBUF)],
        [pltpu.SemaphoreType.DMA for _ in range(NBUF)],
        [pltpu.SemaphoreType.DMA for _ in range(NBUF)],
    ]
    return pl.kernel(
        _gather_body,
        out_type=(
            jax.ShapeDtypeStruct((2 * E, ND), jnp.float32),
            jax.ShapeDtypeStruct((2 * E, XD), jnp.float32),
        ),
        mesh=_mesh(),
        scratch_types=scratch,
        compiler_params=pltpu.CompilerParams(use_tc_tiling_on_sc=False),
    )(h, tx, flat_idx)


def _scatter_body(m_hbm, cw_hbm, dst_hbm, zh_hbm, zx_hbm, ph_hbm, px_hbm,
                  acc_h, acc_x, idx_all, mbs, cbs, gsm, gsc):
    cid = lax.axis_index("c")
    sid = lax.axis_index("s")
    wid = sid * NC + cid
    w_base = wid * S_PER_W
    row0 = sid * ROWS_PER_TILE

    _stage_rows(zh_hbm, acc_h, row0, mbs[0], CHS)
    _stage_rows(zx_hbm, acc_x, row0, cbs[0], CHS)
    w_step0 = wid * S_STEPS
    pltpu.sync_copy(dst_hbm.at[pl.ds(w_step0, S_STEPS)], idx_all)
    plsc.subcore_barrier()

    def group(g, carry):
        k0 = g * NBUF
        for b in range(NBUF):
            base = w_base + (k0 + b) * CHS
            pltpu.async_copy(m_hbm.at[pl.ds(base, CHS)], mbs[b], gsm[b])
            pltpu.async_copy(cw_hbm.at[pl.ds(base, CHS)], cbs[b], gsc[b])
        for b in range(NBUF):
            k = k0 + b
            base = w_base + k * CHS
            pltpu.make_async_copy(m_hbm.at[pl.ds(base, CHS)], mbs[b],
                                  gsm[b]).wait()
            pltpu.make_async_copy(cw_hbm.at[pl.ds(base, CHS)], cbs[b],
                                  gsc[b]).wait()
            pltpu.sync_copy(mbs[b], acc_h.at[idx_all.at[k]], add=True)
            pltpu.sync_copy(cbs[b], acc_x.at[idx_all.at[k]], add=True)
        return carry

    lax.fori_loop(0, S_GROUPS, group, 0)
    plsc.subcore_barrier()

    n_full = ROWS_PER_TILE // CHS
    rem = ROWS_PER_TILE - n_full * CHS
    for j in range(n_full + (1 if rem else 0)):
        sz = CHS if j < n_full else rem
        r = row0 + j * CHS
        pltpu.sync_copy(acc_h.at[pl.ds(r, sz)], mbs[0].at[pl.ds(0, sz)])
        pltpu.sync_copy(mbs[0].at[pl.ds(0, sz)], ph_hbm.at[cid, pl.ds(r, sz)])
        pltpu.sync_copy(acc_x.at[pl.ds(r, sz)], cbs[0].at[pl.ds(0, sz)])
        pltpu.sync_copy(cbs[0].at[pl.ds(0, sz)], px_hbm.at[cid, pl.ds(r, sz)])


def _sc_scatter(m, cwdir, dst, zh, zx):
    scratch = [
        pltpu.VMEM_SHARED((N, ND), jnp.float32),
        pltpu.VMEM_SHARED((N, XD), jnp.float32),
        pltpu.VMEM((S_STEPS, CHS), jnp.int32),
        [pltpu.VMEM((CHS, ND), jnp.float32) for _ in range(NBUF)],
        [pltpu.VMEM((CHS, XD), jnp.float32) for _ in range(NBUF)],
        [pltpu.SemaphoreType.DMA for _ in range(NBUF)],
        [pltpu.SemaphoreType.DMA for _ in range(NBUF)],
    ]
    return pl.kernel(
        _scatter_body,
        out_type=(
            jax.ShapeDtypeStruct((NC, N, ND), jnp.float32),
            jax.ShapeDtypeStruct((NC, N, XD), jnp.float32),
        ),
        mesh=_mesh(),
        scratch_types=scratch,
        compiler_params=pltpu.CompilerParams(use_tc_tiling_on_sc=False),
    )(m, cwdir, dst, zh, zx)


BE = 1280


def _edge_body(dist_ref, hs_ref, hd_ref, xs_ref, xd_ref,
               whs_ref, whd_ref, we_ref, b1_ref, w2n_ref, b2n_ref, w2cb_ref,
               sel_ref, selt_ref,
               ew1t_ref, eb1t_ref, ew2_ref, eb2t_ref,
               m_ref, cw_ref):
    pr = lax.Precision.DEFAULT
    distrow = dist_ref[0]
    pe = _silu(ew1t_ref[...] * distrow + eb1t_ref[...])
    eat = lax.dot_general(ew2_ref[...], pe, (((0,), (0,)), ((), ())),
                          precision=pr, preferred_element_type=jnp.float32)
    eat = eat + eb2t_ref[...]
    pre = lax.dot_general(hs_ref[...], whs_ref[...], (((1,), (0,)), ((), ())),
                          precision=pr, preferred_element_type=jnp.float32)
    pre = pre + lax.dot_general(hd_ref[...], whd_ref[...],
                                (((1,), (0,)), ((), ())),
                                precision=pr, preferred_element_type=jnp.float32)
    pre = pre + lax.dot_general(eat, we_ref[...], (((0,), (0,)), ((), ())),
                                precision=pr, preferred_element_type=jnp.float32)
    act = _silu(pre + b1_ref[...])
    m = lax.dot_general(act[:, :ND], w2n_ref[...], (((1,), (0,)), ((), ())),
                        precision=pr, preferred_element_type=jnp.float32)
    m_ref[...] = m + b2n_ref[...]
    actc = act[:, ND:].reshape(BE // 8, 8 * ND)
    cwg = lax.dot_general(actc, w2cb_ref[...], (((1,), (0,)), ((), ())),
                          precision=pr, preferred_element_type=jnp.float32)
    dP = xs_ref[...] - xd_ref[...]
    l2g = lax.dot_general(dP * dP, sel_ref[...], (((1,), (0,)), ((), ())),
                          precision=pr, preferred_element_type=jnp.float32)
    lng = jnp.maximum(jnp.sqrt(l2g), 1e-8)
    scaleP = lax.dot_general(cwg / lng, selt_ref[...],
                             (((1,), (0,)), ((), ())),
                             precision=pr, preferred_element_type=jnp.float32)
    cw_ref[...] = dP * scaleP


def _tc_edges(distrows, gh, gxp, whs, whd, we, b1, w2n, b2n, w2cb, sel,
              selt, ew1t, eb1t, ew2, eb2t):
    grid = (E // BE,)
    full = lambda shape: pl.BlockSpec(shape, lambda i: (0, 0))
    return pl.pallas_call(
        _edge_body,
        grid=grid,
        in_specs=[
            pl.BlockSpec((1, 1, BE), lambda i: (i, 0, 0)),
            pl.BlockSpec((BE, ND), lambda i: (i, 0)),
            pl.BlockSpec((BE, ND), lambda i: (E // BE + i, 0)),
            pl.BlockSpec((BE // 8, ND), lambda i: (i, 0)),
            pl.BlockSpec((BE // 8, ND), lambda i: (E // BE + i, 0)),
            full((ND, 2 * ND)),
            full((ND, 2 * ND)),
            full((16, 2 * ND)),
            full((1, 2 * ND)),
            full((ND, ND)),
            full((1, ND)),
            full((8 * ND, 8)),
            full((ND, 8)),
            full((8, ND)),
            full((16, 1)),
            full((16, 1)),
            full((16, 16)),
            full((16, 1)),
        ],
        out_specs=[
            pl.BlockSpec((BE, ND), lambda i: (i, 0)),
            pl.BlockSpec((BE // 8, ND), lambda i: (i, 0)),
        ],
        out_shape=[
            jax.ShapeDtypeStruct((E, ND), jnp.float32),
            jax.ShapeDtypeStruct((E // 8, ND), jnp.float32),
        ],
        compiler_params=pltpu.CompilerParams(
            dimension_semantics=("arbitrary",)),
    )(distrows, gh, gh, gxp, gxp, whs, whd, we, b1, w2n, b2n, w2cb, sel, selt,
      ew1t, eb1t, ew2, eb2t)


BN = 2000


def _combine_body(h_ref, tx_ref, pha_ref, phb_ref, pxa_ref, pxb_ref,
                  oh_ref, ox_ref):
    oh_ref[...] = h_ref[...] + pha_ref[0] + phb_ref[0]
    ox_ref[...] = tx_ref[...] + pxa_ref[0] + pxb_ref[0]


def _tc_combine(h, tx, ph, px):
    grid = (N // BN,)
    return pl.pallas_call(
        _combine_body,
        grid=grid,
        in_specs=[
            pl.BlockSpec((BN, ND), lambda i: (i, 0)),
            pl.BlockSpec((BN, XD), lambda i: (i, 0)),
            pl.BlockSpec((1, BN, ND), lambda i: (0, i, 0)),
            pl.BlockSpec((1, BN, ND), lambda i: (1, i, 0)),
            pl.BlockSpec((1, BN, XD), lambda i: (0, i, 0)),
            pl.BlockSpec((1, BN, XD), lambda i: (1, i, 0)),
        ],
        out_specs=[
            pl.BlockSpec((BN, ND), lambda i: (i, 0)),
            pl.BlockSpec((BN, XD), lambda i: (i, 0)),
        ],
        out_shape=[
            jax.ShapeDtypeStruct((N, ND), jnp.float32),
            jax.ShapeDtypeStruct((N, XD), jnp.float32),
        ],
        compiler_params=pltpu.CompilerParams(
            dimension_semantics=("arbitrary",)),
    )(h, tx, ph, ph, px, px)


@jax.jit
def kernel(h, x, edge_idx, edge_dist,
           node_W1, node_b1, node_W2, node_b2,
           coord_W1, coord_b1, coord_W2,
           edge_W1, edge_b1, edge_W2, edge_b2):
    f32 = jnp.float32
    tx = jnp.pad(x, ((0, 0), (0, XD - 3)))
    flat_idx = edge_idx.reshape(2 * E // CH, CH)
    dst = edge_idx[1].reshape(E // CHS, CHS)
    distrows = edge_dist.reshape(E // BE, 1, BE)

    whs = jnp.concatenate([node_W1[:ND], coord_W1[:ND]], axis=1)
    whd = jnp.concatenate([node_W1[ND:2 * ND], coord_W1[ND:2 * ND]], axis=1)
    we = jnp.concatenate([node_W1[2 * ND:], coord_W1[2 * ND:]], axis=1)
    b1 = jnp.concatenate([node_b1, coord_b1]).reshape(1, 2 * ND)
    b2n = node_b2.reshape(1, ND)
    ew1t = edge_W1.reshape(16, 1)
    eb1t = edge_b1.reshape(16, 1)
    eb2t = edge_b2.reshape(16, 1)
    zh = jnp.zeros((N, ND), f32)
    zx = jnp.zeros((N, XD), f32)

    w2cb = jnp.kron(jnp.eye(8, dtype=f32), coord_W2)
    lane_grp = jnp.arange(ND, dtype=jnp.int32) // XD
    sel = (lane_grp[:, None] == jnp.arange(8)[None, :]).astype(f32)
    selt = sel.T

    gh, gx = _sc_gather(h, tx, flat_idx)
    gxp = gx.reshape(2 * E // 8, ND)
    m, cwp = _tc_edges(distrows, gh, gxp, whs, whd, we, b1,
                       node_W2, b2n, w2cb, sel, selt, ew1t, eb1t,
                       edge_W2, eb2t)
    cwdir = cwp.reshape(E, XD)
    ph, px = _sc_scatter(m, cwdir, dst, zh, zx)
    oh, ox = _tc_combine(h, tx, ph, px)
    return oh, ox[:, :3]


# speedup vs baseline: 2.4116x; 1.0473x over previous
import jax
import jax.numpy as jnp
from jax import lax
from jax.experimental import pallas as pl
from jax.experimental.pallas import tpu as pltpu
from jax.experimental.pallas import tpu_sc as plsc

N = 10000
E = 320000
ND = 128
XD = 16

NC = 2
NS = 16
NW = NC * NS

CH = 80
CHS = 40
NBUF = 5
G_PER_W = (2 * E) // NW
G_STEPS = G_PER_W // CH
G_GROUPS = G_STEPS // NBUF
S_PER_W = E // NW
S_STEPS = S_PER_W // CHS
S_GROUPS = S_STEPS // NBUF

ROWS_PER_TILE = N // NS


def _silu(v):
    return v * jax.nn.sigmoid(v)


def _mesh():
    return plsc.VectorSubcoreMesh(core_axis_name="c", subcore_axis_name="s",
                                  num_cores=NC, num_subcores=NS)


def _stage_rows(src_hbm, dst_sh, row0, bounce, cap):
    n_full = ROWS_PER_TILE // cap
    rem = ROWS_PER_TILE - n_full * cap
    for j in range(n_full + (1 if rem else 0)):
        sz = cap if j < n_full else rem
        r = row0 + j * cap
        pltpu.sync_copy(src_hbm.at[pl.ds(r, sz)], bounce.at[pl.ds(0, sz)])
        pltpu.sync_copy(bounce.at[pl.ds(0, sz)], dst_sh.at[pl.ds(r, sz)])


def _gather_body(h_hbm, tx_hbm, idx_hbm, gh_hbm, gx_hbm,
                 idx_all, hbs, xbs, gsh, gsx, osh, osx):
    cid = lax.axis_index("c")
    sid = lax.axis_index("s")
    wid = sid * NC + cid
    w_base = wid * G_PER_W

    w_step0 = wid * G_STEPS
    pltpu.sync_copy(idx_hbm.at[pl.ds(w_step0, G_STEPS)], idx_all)
    plsc.subcore_barrier()

    def group(g, carry):
        k0 = g * NBUF
        for b in range(NBUF):
            k = k0 + b
            pltpu.async_copy(h_hbm.at[idx_all.at[k]], hbs[b], gsh[b])
            pltpu.async_copy(tx_hbm.at[idx_all.at[k]], xbs[b], gsx[b])
        for b in range(NBUF):
            k = k0 + b
            base = w_base + k * CH
            pltpu.make_async_copy(h_hbm.at[idx_all.at[k]], hbs[b],
                                  gsh[b]).wait()
            pltpu.make_async_copy(tx_hbm.at[idx_all.at[k]], xbs[b],
                                  gsx[b]).wait()
            pltpu.async_copy(hbs[b], gh_hbm.at[pl.ds(base, CH)], osh[b])
            pltpu.async_copy(xbs[b], gx_hbm.at[pl.ds(base, CH)], osx[b])
        for b in range(NBUF):
            base = w_base + (k0 + b) * CH
            pltpu.make_async_copy(hbs[b], gh_hbm.at[pl.ds(base, CH)],
                                  osh[b]).wait()
            pltpu.make_async_copy(xbs[b], gx_hbm.at[pl.ds(base, CH)],
                                  osx[b]).wait()
        return carry

    lax.fori_loop(0, G_GROUPS, group, 0)


def _sc_gather(h, tx, flat_idx):
    scratch = [
        pltpu.VMEM((G_STEPS, CH), jnp.int32),
        [pltpu.VMEM((CH, ND), jnp.float32) for _ in range(NBUF)],
        [pltpu.VMEM((CH, XD), jnp.float32) for _ in range(NBUF)],
        [pltpu.SemaphoreType.DMA for _ in range(NBUF)],
        [pltpu.SemaphoreType.DMA for _ in range(NBUF)],
        [pltpu.SemaphoreType.DMA for _ in range(NBUF)],
        [pltpu.SemaphoreType.DMA for _ in range(NBUF)],
    ]
    return pl.kernel(
        _gather_body,
        out_type=(
            jax.ShapeDtypeStruct((2 * E, ND), jnp.float32),
            jax.ShapeDtypeStruct((2 * E, XD), jnp.float32),
        ),
        mesh=_mesh(),
        scratch_types=scratch,
        compiler_params=pltpu.CompilerParams(use_tc_tiling_on_sc=False),
    )(h, tx, flat_idx)


def _scatter_body(m_hbm, cw_hbm, dst_hbm, zh_hbm, zx_hbm, ph_hbm, px_hbm,
                  acc_h, acc_x, idx_all, mbs, cbs, gsm, gsc):
    cid = lax.axis_index("c")
    sid = lax.axis_index("s")
    wid = sid * NC + cid
    w_base = wid * S_PER_W
    row0 = sid * ROWS_PER_TILE

    _stage_rows(zh_hbm, acc_h, row0, mbs[0], CHS)
    _stage_rows(zx_hbm, acc_x, row0, cbs[0], CHS)
    w_step0 = wid * S_STEPS
    pltpu.sync_copy(dst_hbm.at[pl.ds(w_step0, S_STEPS)], idx_all)
    plsc.subcore_barrier()

    def group(g, carry):
        k0 = g * NBUF
        for b in range(NBUF):
            base = w_base + (k0 + b) * CHS
            pltpu.async_copy(m_hbm.at[pl.ds(base, CHS)], mbs[b], gsm[b])
            pltpu.async_copy(cw_hbm.at[pl.ds(base, CHS)], cbs[b], gsc[b])
        for b in range(NBUF):
            k = k0 + b
            base = w_base + k * CHS
            pltpu.make_async_copy(m_hbm.at[pl.ds(base, CHS)], mbs[b],
                                  gsm[b]).wait()
            pltpu.make_async_copy(cw_hbm.at[pl.ds(base, CHS)], cbs[b],
                                  gsc[b]).wait()
            pltpu.sync_copy(mbs[b], acc_h.at[idx_all.at[k]], add=True)
            pltpu.sync_copy(cbs[b], acc_x.at[idx_all.at[k]], add=True)
        return carry

    lax.fori_loop(0, S_GROUPS, group, 0)
    plsc.subcore_barrier()

    n_full = ROWS_PER_TILE // CHS
    rem = ROWS_PER_TILE - n_full * CHS
    for j in range(n_full + (1 if rem else 0)):
        sz = CHS if j < n_full else rem
        r = row0 + j * CHS
        pltpu.sync_copy(acc_h.at[pl.ds(r, sz)], mbs[0].at[pl.ds(0, sz)])
        pltpu.sync_copy(mbs[0].at[pl.ds(0, sz)], ph_hbm.at[cid, pl.ds(r, sz)])
        pltpu.sync_copy(acc_x.at[pl.ds(r, sz)], cbs[0].at[pl.ds(0, sz)])
        pltpu.sync_copy(cbs[0].at[pl.ds(0, sz)], px_hbm.at[cid, pl.ds(r, sz)])


def _sc_scatter(m, cwdir, dst, zh, zx):
    scratch = [
        pltpu.VMEM_SHARED((N, ND), jnp.float32),
        pltpu.VMEM_SHARED((N, XD), jnp.float32),
        pltpu.VMEM((S_STEPS, CHS), jnp.int32),
        [pltpu.VMEM((CHS, ND), jnp.float32) for _ in range(NBUF)],
        [pltpu.VMEM((CHS, XD), jnp.float32) for _ in range(NBUF)],
        [pltpu.SemaphoreType.DMA for _ in range(NBUF)],
        [pltpu.SemaphoreType.DMA for _ in range(NBUF)],
    ]
    return pl.kernel(
        _scatter_body,
        out_type=(
            jax.ShapeDtypeStruct((NC, N, ND), jnp.float32),
            jax.ShapeDtypeStruct((NC, N, XD), jnp.float32),
        ),
        mesh=_mesh(),
        scratch_types=scratch,
        compiler_params=pltpu.CompilerParams(use_tc_tiling_on_sc=False),
    )(m, cwdir, dst, zh, zx)


BE = 2560


def _edge_body(dist_ref, hs_ref, hd_ref, xs_ref, xd_ref,
               whs_ref, whd_ref, we_ref, b1_ref, w2n_ref, b2n_ref, w2cb_ref,
               sel_ref, selt_ref,
               ew1t_ref, eb1t_ref, ew2_ref, eb2t_ref,
               m_ref, cw_ref):
    pr = lax.Precision.DEFAULT
    distrow = dist_ref[0]
    pe = _silu(ew1t_ref[...] * distrow + eb1t_ref[...])
    eat = lax.dot_general(ew2_ref[...], pe, (((0,), (0,)), ((), ())),
                          precision=pr, preferred_element_type=jnp.float32)
    eat = eat + eb2t_ref[...]
    pre = lax.dot_general(hs_ref[...], whs_ref[...], (((1,), (0,)), ((), ())),
                          precision=pr, preferred_element_type=jnp.float32)
    pre = pre + lax.dot_general(hd_ref[...], whd_ref[...],
                                (((1,), (0,)), ((), ())),
                                precision=pr, preferred_element_type=jnp.float32)
    pre = pre + lax.dot_general(eat, we_ref[...], (((0,), (0,)), ((), ())),
                                precision=pr, preferred_element_type=jnp.float32)
    act = _silu(pre + b1_ref[...])
    m = lax.dot_general(act[:, :ND], w2n_ref[...], (((1,), (0,)), ((), ())),
                        precision=pr, preferred_element_type=jnp.float32)
    m_ref[...] = m + b2n_ref[...]
    actc = act[:, ND:].reshape(BE // 8, 8 * ND)
    cwg = lax.dot_general(actc, w2cb_ref[...], (((1,), (0,)), ((), ())),
                          precision=pr, preferred_element_type=jnp.float32)
    dP = xs_ref[...] - xd_ref[...]
    l2g = lax.dot_general(dP * dP, sel_ref[...], (((1,), (0,)), ((), ())),
                          precision=pr, preferred_element_type=jnp.float32)
    lng = jnp.maximum(jnp.sqrt(l2g), 1e-8)
    scaleP = lax.dot_general(cwg / lng, selt_ref[...],
                             (((1,), (0,)), ((), ())),
                             precision=pr, preferred_element_type=jnp.float32)
    cw_ref[...] = dP * scaleP


def _tc_edges(distrows, gh, gxp, whs, whd, we, b1, w2n, b2n, w2cb, sel,
              selt, ew1t, eb1t, ew2, eb2t):
    grid = (E // BE,)
    full = lambda shape: pl.BlockSpec(shape, lambda i: (0, 0))
    return pl.pallas_call(
        _edge_body,
        grid=grid,
        in_specs=[
            pl.BlockSpec((1, 1, BE), lambda i: (i, 0, 0)),
            pl.BlockSpec((BE, ND), lambda i: (i, 0)),
            pl.BlockSpec((BE, ND), lambda i: (E // BE + i, 0)),
            pl.BlockSpec((BE // 8, ND), lambda i: (i, 0)),
            pl.BlockSpec((BE // 8, ND), lambda i: (E // BE + i, 0)),
            full((ND, 2 * ND)),
            full((ND, 2 * ND)),
            full((16, 2 * ND)),
            full((1, 2 * ND)),
            full((ND, ND)),
            full((1, ND)),
            full((8 * ND, 8)),
            full((ND, 8)),
            full((8, ND)),
            full((16, 1)),
            full((16, 1)),
            full((16, 16)),
            full((16, 1)),
        ],
        out_specs=[
            pl.BlockSpec((BE, ND), lambda i: (i, 0)),
            pl.BlockSpec((BE // 8, ND), lambda i: (i, 0)),
        ],
        out_shape=[
            jax.ShapeDtypeStruct((E, ND), jnp.float32),
            jax.ShapeDtypeStruct((E // 8, ND), jnp.float32),
        ],
        compiler_params=pltpu.CompilerParams(
            dimension_semantics=("arbitrary",)),
    )(distrows, gh, gh, gxp, gxp, whs, whd, we, b1, w2n, b2n, w2cb, sel, selt,
      ew1t, eb1t, ew2, eb2t)


BN = 2000


def _combine_body(h_ref, tx_ref, pha_ref, phb_ref, pxa_ref, pxb_ref,
                  oh_ref, ox_ref):
    oh_ref[...] = h_ref[...] + pha_ref[0] + phb_ref[0]
    ox_ref[...] = tx_ref[...] + pxa_ref[0] + pxb_ref[0]


def _tc_combine(h, tx, ph, px):
    grid = (N // BN,)
    return pl.pallas_call(
        _combine_body,
        grid=grid,
        in_specs=[
            pl.BlockSpec((BN, ND), lambda i: (i, 0)),
            pl.BlockSpec((BN, XD), lambda i: (i, 0)),
            pl.BlockSpec((1, BN, ND), lambda i: (0, i, 0)),
            pl.BlockSpec((1, BN, ND), lambda i: (1, i, 0)),
            pl.BlockSpec((1, BN, XD), lambda i: (0, i, 0)),
            pl.BlockSpec((1, BN, XD), lambda i: (1, i, 0)),
        ],
        out_specs=[
            pl.BlockSpec((BN, ND), lambda i: (i, 0)),
            pl.BlockSpec((BN, XD), lambda i: (i, 0)),
        ],
        out_shape=[
            jax.ShapeDtypeStruct((N, ND), jnp.float32),
            jax.ShapeDtypeStruct((N, XD), jnp.float32),
        ],
        compiler_params=pltpu.CompilerParams(
            dimension_semantics=("arbitrary",)),
    )(h, tx, ph, ph, px, px)


@jax.jit
def kernel(h, x, edge_idx, edge_dist,
           node_W1, node_b1, node_W2, node_b2,
           coord_W1, coord_b1, coord_W2,
           edge_W1, edge_b1, edge_W2, edge_b2):
    f32 = jnp.float32
    tx = jnp.pad(x, ((0, 0), (0, XD - 3)))
    flat_idx = edge_idx.reshape(2 * E // CH, CH)
    dst = edge_idx[1].reshape(E // CHS, CHS)
    distrows = edge_dist.reshape(E // BE, 1, BE)

    whs = jnp.concatenate([node_W1[:ND], coord_W1[:ND]], axis=1)
    whd = jnp.concatenate([node_W1[ND:2 * ND], coord_W1[ND:2 * ND]], axis=1)
    we = jnp.concatenate([node_W1[2 * ND:], coord_W1[2 * ND:]], axis=1)
    b1 = jnp.concatenate([node_b1, coord_b1]).reshape(1, 2 * ND)
    b2n = node_b2.reshape(1, ND)
    ew1t = edge_W1.reshape(16, 1)
    eb1t = edge_b1.reshape(16, 1)
    eb2t = edge_b2.reshape(16, 1)
    zh = jnp.zeros((N, ND), f32)
    zx = jnp.zeros((N, XD), f32)

    w2cb = jnp.kron(jnp.eye(8, dtype=f32), coord_W2)
    lane_grp = jnp.arange(ND, dtype=jnp.int32) // XD
    sel = (lane_grp[:, None] == jnp.arange(8)[None, :]).astype(f32)
    selt = sel.T

    gh, gx = _sc_gather(h, tx, flat_idx)
    gxp = gx.reshape(2 * E // 8, ND)
    m, cwp = _tc_edges(distrows, gh, gxp, whs, whd, we, b1,
                       node_W2, b2n, w2cb, sel, selt, ew1t, eb1t,
                       edge_W2, eb2t)
    cwdir = cwp.reshape(E, XD)
    ph, px = _sc_scatter(m, cwdir, dst, zh, zx)
    oh, ox = _tc_combine(h, tx, ph, px)
    return oh, ox[:, :3]
